# baseline (speedup 1.0000x reference)
import functools

import jax
import jax.numpy as jnp
from jax import lax
from jax.experimental import pallas as pl
from jax.experimental.pallas import tpu as pltpu
from jax.experimental.pallas import tpu_sc as plsc

E = 8
TOPK = 2
D = 2048
DFF = 4096
T = 8192
TE = T // E


def _fc1_body(x_ref, w1_ref, h_ref, xb_ref):
    f = pl.program_id(1)

    @pl.when(f == 0)
    def _():
        xb_ref[...] = x_ref[...].astype(jnp.bfloat16)

    w1b = w1_ref[0].astype(jnp.bfloat16)
    h = lax.dot_general(xb_ref[...], w1b, (((1,), (1,)), ((), ())),
                        preferred_element_type=jnp.float32)
    h_ref[...] = jax.nn.gelu(h).astype(jnp.bfloat16)


def _fc2_body(h_ref, gate_ref, w2_ref, out_ref):
    w2b = w2_ref[0].astype(jnp.bfloat16)
    y = lax.dot_general(h_ref[...], w2b, (((1,), (1,)), ((), ())),
                        preferred_element_type=jnp.float32)
    out_ref[...] = y * gate_ref[...]


BF = 512
NF1 = DFF // BF
BD = 512
ND = D // BD


def _expert_mlp(x, gate2d, W1, W2):
    h = pl.pallas_call(
        _fc1_body,
        grid=(E, NF1),
        in_specs=[
            pl.BlockSpec((TE, D), lambda e, f: (e, 0)),
            pl.BlockSpec((1, BF, D), lambda e, f: (e, f, 0)),
        ],
        out_specs=pl.BlockSpec((TE, BF), lambda e, f: (e, f)),
        out_shape=jax.ShapeDtypeStruct((T, DFF), jnp.bfloat16),
        scratch_shapes=[pltpu.VMEM((TE, D), jnp.bfloat16)],
        compiler_params=pltpu.CompilerParams(
            dimension_semantics=("parallel", "arbitrary")),
    )(x, W1)
    return pl.pallas_call(
        _fc2_body,
        grid=(E, ND),
        in_specs=[
            pl.BlockSpec((TE, DFF), lambda e, d: (e, 0)),
            pl.BlockSpec((TE, 1), lambda e, d: (e, 0)),
            pl.BlockSpec((1, BD, DFF), lambda e, d: (e, d, 0)),
        ],
        out_specs=pl.BlockSpec((TE, BD), lambda e, d: (e, d)),
        out_shape=jax.ShapeDtypeStruct((T, D), jnp.float32),
        compiler_params=pltpu.CompilerParams(
            dimension_semantics=("parallel", "arbitrary")),
    )(h, gate2d, W2)


NW = 32
IPW = T // NW
CH = 16
NCHK = IPW // CH


def _scatter_body(res_hbm, nidx_hbm, full_hbm, nidx_v, idx_v, bufA, bufB,
                  lsA, lsB, ssA, ssB):
    w = lax.axis_index("c") * 16 + lax.axis_index("s")
    base = w * IPW
    pltpu.sync_copy(nidx_hbm.at[pl.ds(base, IPW)], nidx_v)
    for i in range(NCHK):
        idx_v[i, ...] = nidx_v[pl.ds(i * CH, CH)]
    bufs = (bufA, bufB)
    lsems = (lsA, lsB)
    ssems = (ssA, ssB)
    loads = [None] * NCHK
    scats = [None] * NCHK
    loads[0] = pltpu.async_copy(
        res_hbm.at[pl.ds(base, CH)], bufs[0], lsems[0])
    for ch in range(NCHK):
        b = ch % 2
        loads[ch].wait()
        scats[ch] = pltpu.async_copy(
            bufs[b], full_hbm.at[idx_v.at[ch]], ssems[b])
        if ch + 1 < NCHK:
            nb = (ch + 1) % 2
            if ch >= 1:
                scats[ch - 1].wait()
            loads[ch + 1] = pltpu.async_copy(
                res_hbm.at[pl.ds(base + (ch + 1) * CH, CH)], bufs[nb],
                lsems[nb])
    scats[NCHK - 2].wait()
    scats[NCHK - 1].wait()


@functools.partial(
    pl.kernel,
    out_type=jax.ShapeDtypeStruct((T, D), jnp.float32),
    mesh=plsc.VectorSubcoreMesh(core_axis_name="c", subcore_axis_name="s"),
    scratch_types=[
        pltpu.VMEM((IPW,), jnp.int32),
        pltpu.VMEM((NCHK, CH), jnp.int32),
        pltpu.VMEM((CH, D), jnp.float32),
        pltpu.VMEM((CH, D), jnp.float32),
        pltpu.SemaphoreType.DMA,
        pltpu.SemaphoreType.DMA,
        pltpu.SemaphoreType.DMA,
        pltpu.SemaphoreType.DMA,
    ],
)
def _scatter(res_hbm, nidx_hbm, full_hbm, nidx_v, idx_v, bufA, bufB,
             lsA, lsB, ssA, ssB):
    _scatter_body(res_hbm, nidx_hbm, full_hbm, nidx_v, idx_v, bufA, bufB,
                  lsA, lsB, ssA, ssB)


BT = 256


def _pairsum_body(in_ref, out_ref):
    out_ref[...] = in_ref[:, 0, :] + in_ref[:, 1, :]


def _pairsum(full3):
    return pl.pallas_call(
        _pairsum_body,
        grid=((T // TOPK) // BT,),
        in_specs=[pl.BlockSpec((BT, TOPK, D), lambda i: (i, 0, 0))],
        out_specs=pl.BlockSpec((BT, D), lambda i: (i, 0)),
        out_shape=jax.ShapeDtypeStruct((T // TOPK, D), jnp.float32),
        compiler_params=pltpu.CompilerParams(
            dimension_semantics=("arbitrary",)),
    )(full3)


def kernel(inputs_shard, gate_weight, choosed_experts, new_index, W1, W2):
    gate2d = gate_weight.reshape(T, 1)
    result = _expert_mlp(inputs_shard, gate2d, W1, W2)
    full = _scatter(result, new_index)
    out2 = _pairsum(full.reshape(T // TOPK, TOPK, D))
    mlp_bias = jnp.zeros((D,), dtype=out2.dtype)
    return (out2, mlp_bias)


# speedup vs baseline: 1.1625x; 1.1625x over previous
import functools

import jax
import jax.numpy as jnp
from jax import lax
from jax.experimental import pallas as pl
from jax.experimental.pallas import tpu as pltpu
from jax.experimental.pallas import tpu_sc as plsc

E = 8
TOPK = 2
D = 2048
DFF = 4096
T = 8192
TE = T // E


def _fc1_body(x_ref, w1_ref, h_ref, xb_ref):
    f = pl.program_id(1)

    @pl.when(f == 0)
    def _():
        xb_ref[...] = x_ref[...].astype(jnp.bfloat16)

    w1b = w1_ref[0].astype(jnp.bfloat16)
    h = lax.dot_general(xb_ref[...], w1b, (((1,), (1,)), ((), ())),
                        preferred_element_type=jnp.float32)
    h_ref[...] = jax.nn.gelu(h).astype(jnp.bfloat16)


def _fc2_body(h_ref, gate_ref, w2_ref, out_ref):
    w2b = w2_ref[0].astype(jnp.bfloat16)
    y = lax.dot_general(h_ref[...], w2b, (((1,), (1,)), ((), ())),
                        preferred_element_type=jnp.float32)
    out_ref[...] = y * gate_ref[...]


BF = 512
NF1 = DFF // BF
BD = 512
ND = D // BD


def _expert_mlp(x, gate2d, W1, W2):
    h = pl.pallas_call(
        _fc1_body,
        grid=(E, NF1),
        in_specs=[
            pl.BlockSpec((TE, D), lambda e, f: (e, 0)),
            pl.BlockSpec((1, BF, D), lambda e, f: (e, f, 0)),
        ],
        out_specs=pl.BlockSpec((TE, BF), lambda e, f: (e, f)),
        out_shape=jax.ShapeDtypeStruct((T, DFF), jnp.bfloat16),
        scratch_shapes=[pltpu.VMEM((TE, D), jnp.bfloat16)],
        compiler_params=pltpu.CompilerParams(
            dimension_semantics=("parallel", "arbitrary")),
    )(x, W1)
    return pl.pallas_call(
        _fc2_body,
        grid=(E, ND),
        in_specs=[
            pl.BlockSpec((TE, DFF), lambda e, d: (e, 0)),
            pl.BlockSpec((TE, 1), lambda e, d: (e, 0)),
            pl.BlockSpec((1, BD, DFF), lambda e, d: (e, d, 0)),
        ],
        out_specs=pl.BlockSpec((TE, BD), lambda e, d: (e, d)),
        out_shape=jax.ShapeDtypeStruct((T, D), jnp.float32),
        compiler_params=pltpu.CompilerParams(
            dimension_semantics=("parallel", "arbitrary")),
    )(h, gate2d, W2)


NW = 32
IPW = T // NW
CH = 16
NCHK = IPW // CH


def _scatter_body(res_hbm, nidx_hbm, full_hbm, nidx_v, idx_v, bufA, bufB,
                  lsA, lsB, ssA, ssB):
    w = lax.axis_index("c") * 16 + lax.axis_index("s")
    base = w * IPW
    pltpu.sync_copy(nidx_hbm.at[pl.ds(base, IPW)], nidx_v)
    for i in range(NCHK):
        v = nidx_v[pl.ds(i * CH, CH)]
        idx_v[i, ...] = ((v & 1) << 12) | lax.shift_right_logical(v, 1)
    bufs = (bufA, bufB)
    lsems = (lsA, lsB)
    ssems = (ssA, ssB)
    loads = [None] * NCHK
    scats = [None] * NCHK
    loads[0] = pltpu.async_copy(
        res_hbm.at[pl.ds(base, CH)], bufs[0], lsems[0])
    for ch in range(NCHK):
        b = ch % 2
        loads[ch].wait()
        scats[ch] = pltpu.async_copy(
            bufs[b], full_hbm.at[idx_v.at[ch]], ssems[b])
        if ch + 1 < NCHK:
            nb = (ch + 1) % 2
            if ch >= 1:
                scats[ch - 1].wait()
            loads[ch + 1] = pltpu.async_copy(
                res_hbm.at[pl.ds(base + (ch + 1) * CH, CH)], bufs[nb],
                lsems[nb])
    scats[NCHK - 2].wait()
    scats[NCHK - 1].wait()


@functools.partial(
    pl.kernel,
    out_type=jax.ShapeDtypeStruct((T, D), jnp.float32),
    mesh=plsc.VectorSubcoreMesh(core_axis_name="c", subcore_axis_name="s"),
    scratch_types=[
        pltpu.VMEM((IPW,), jnp.int32),
        pltpu.VMEM((NCHK, CH), jnp.int32),
        pltpu.VMEM((CH, D), jnp.float32),
        pltpu.VMEM((CH, D), jnp.float32),
        pltpu.SemaphoreType.DMA,
        pltpu.SemaphoreType.DMA,
        pltpu.SemaphoreType.DMA,
        pltpu.SemaphoreType.DMA,
    ],
)
def _scatter(res_hbm, nidx_hbm, full_hbm, nidx_v, idx_v, bufA, bufB,
             lsA, lsB, ssA, ssB):
    _scatter_body(res_hbm, nidx_hbm, full_hbm, nidx_v, idx_v, bufA, bufB,
                  lsA, lsB, ssA, ssB)


BT = 256
NBT = (T // TOPK) // BT


def _pairsum_body(a_ref, b_ref, out_ref):
    out_ref[...] = a_ref[...] + b_ref[...]


def _pairsum(full):
    return pl.pallas_call(
        _pairsum_body,
        grid=(NBT,),
        in_specs=[
            pl.BlockSpec((BT, D), lambda i: (i, 0)),
            pl.BlockSpec((BT, D), lambda i: (i + NBT, 0)),
        ],
        out_specs=pl.BlockSpec((BT, D), lambda i: (i, 0)),
        out_shape=jax.ShapeDtypeStruct((T // TOPK, D), jnp.float32),
        compiler_params=pltpu.CompilerParams(
            dimension_semantics=("arbitrary",)),
    )(full, full)


def kernel(inputs_shard, gate_weight, choosed_experts, new_index, W1, W2):
    gate2d = gate_weight.reshape(T, 1)
    result = _expert_mlp(inputs_shard, gate2d, W1, W2)
    full = _scatter(result, new_index)
    out2 = _pairsum(full)
    mlp_bias = jnp.zeros((D,), dtype=out2.dtype)
    return (out2, mlp_bias)


# speedup vs baseline: 1.1786x; 1.0139x over previous
import functools

import jax
import jax.numpy as jnp
from jax import lax
from jax.experimental import pallas as pl
from jax.experimental.pallas import tpu as pltpu
from jax.experimental.pallas import tpu_sc as plsc

E = 8
TOPK = 2
D = 2048
DFF = 4096
T = 8192
TE = T // E


def _fc1_body(x_ref, w1_ref, h_ref):
    h = lax.dot_general(x_ref[...], w1_ref[0], (((1,), (1,)), ((), ())),
                        preferred_element_type=jnp.float32)
    h_ref[...] = jax.nn.gelu(h).astype(jnp.bfloat16)


def _fc2_body(h_ref, gate_ref, w2_ref, out_ref):
    y = lax.dot_general(h_ref[...], w2_ref[0], (((1,), (1,)), ((), ())),
                        preferred_element_type=jnp.float32)
    out_ref[...] = y * gate_ref[...]


BF = 512
NF1 = DFF // BF
BD = 512
ND = D // BD


def _expert_mlp(x, gate2d, W1, W2):
    h = pl.pallas_call(
        _fc1_body,
        grid=(E, NF1),
        in_specs=[
            pl.BlockSpec((TE, D), lambda e, f: (e, 0)),
            pl.BlockSpec((1, BF, D), lambda e, f: (e, f, 0)),
        ],
        out_specs=pl.BlockSpec((TE, BF), lambda e, f: (e, f)),
        out_shape=jax.ShapeDtypeStruct((T, DFF), jnp.bfloat16),
        compiler_params=pltpu.CompilerParams(
            dimension_semantics=("parallel", "arbitrary")),
    )(x, W1)
    return pl.pallas_call(
        _fc2_body,
        grid=(E, ND),
        in_specs=[
            pl.BlockSpec((TE, DFF), lambda e, d: (e, 0)),
            pl.BlockSpec((TE, 1), lambda e, d: (e, 0)),
            pl.BlockSpec((1, BD, DFF), lambda e, d: (e, d, 0)),
        ],
        out_specs=pl.BlockSpec((TE, BD), lambda e, d: (e, d)),
        out_shape=jax.ShapeDtypeStruct((T, D), jnp.float32),
        compiler_params=pltpu.CompilerParams(
            dimension_semantics=("parallel", "arbitrary")),
    )(h, gate2d, W2)


NW = 32
IPW = T // NW
CH = 16
NCHK = IPW // CH


def _scatter_body(res_hbm, nidx_hbm, full_hbm, nidx_v, idx_v, bufA, bufB,
                  lsA, lsB, ssA, ssB):
    w = lax.axis_index("c") * 16 + lax.axis_index("s")
    base = w * IPW
    pltpu.sync_copy(nidx_hbm.at[pl.ds(base, IPW)], nidx_v)
    for i in range(NCHK):
        v = nidx_v[pl.ds(i * CH, CH)]
        idx_v[i, ...] = ((v & 1) << 12) | lax.shift_right_logical(v, 1)
    bufs = (bufA, bufB)
    lsems = (lsA, lsB)
    ssems = (ssA, ssB)
    loads = [None] * NCHK
    scats = [None] * NCHK
    loads[0] = pltpu.async_copy(
        res_hbm.at[pl.ds(base, CH)], bufs[0], lsems[0])
    for ch in range(NCHK):
        b = ch % 2
        loads[ch].wait()
        scats[ch] = pltpu.async_copy(
            bufs[b], full_hbm.at[idx_v.at[ch]], ssems[b])
        if ch + 1 < NCHK:
            nb = (ch + 1) % 2
            if ch >= 1:
                scats[ch - 1].wait()
            loads[ch + 1] = pltpu.async_copy(
                res_hbm.at[pl.ds(base + (ch + 1) * CH, CH)], bufs[nb],
                lsems[nb])
    scats[NCHK - 2].wait()
    scats[NCHK - 1].wait()


@functools.partial(
    pl.kernel,
    out_type=jax.ShapeDtypeStruct((T, D), jnp.float32),
    mesh=plsc.VectorSubcoreMesh(core_axis_name="c", subcore_axis_name="s"),
    scratch_types=[
        pltpu.VMEM((IPW,), jnp.int32),
        pltpu.VMEM((NCHK, CH), jnp.int32),
        pltpu.VMEM((CH, D), jnp.float32),
        pltpu.VMEM((CH, D), jnp.float32),
        pltpu.SemaphoreType.DMA,
        pltpu.SemaphoreType.DMA,
        pltpu.SemaphoreType.DMA,
        pltpu.SemaphoreType.DMA,
    ],
)
def _scatter(res_hbm, nidx_hbm, full_hbm, nidx_v, idx_v, bufA, bufB,
             lsA, lsB, ssA, ssB):
    _scatter_body(res_hbm, nidx_hbm, full_hbm, nidx_v, idx_v, bufA, bufB,
                  lsA, lsB, ssA, ssB)


BT = 256
NBT = (T // TOPK) // BT


def _pairsum_body(a_ref, b_ref, out_ref):
    out_ref[...] = a_ref[...] + b_ref[...]


def _pairsum(full):
    return pl.pallas_call(
        _pairsum_body,
        grid=(NBT,),
        in_specs=[
            pl.BlockSpec((BT, D), lambda i: (i, 0)),
            pl.BlockSpec((BT, D), lambda i: (i + NBT, 0)),
        ],
        out_specs=pl.BlockSpec((BT, D), lambda i: (i, 0)),
        out_shape=jax.ShapeDtypeStruct((T // TOPK, D), jnp.float32),
        compiler_params=pltpu.CompilerParams(
            dimension_semantics=("arbitrary",)),
    )(full, full)


def kernel(inputs_shard, gate_weight, choosed_experts, new_index, W1, W2):
    gate2d = gate_weight.reshape(T, 1)
    result = _expert_mlp(inputs_shard, gate2d, W1, W2)
    full = _scatter(result, new_index)
    out2 = _pairsum(full)
    mlp_bias = jnp.zeros((D,), dtype=out2.dtype)
    return (out2, mlp_bias)


# speedup vs baseline: 1.2492x; 1.0599x over previous
import functools

import jax
import jax.numpy as jnp
from jax import lax
from jax.experimental import pallas as pl
from jax.experimental.pallas import tpu as pltpu
from jax.experimental.pallas import tpu_sc as plsc

E = 8
TOPK = 2
D = 2048
DFF = 4096
T = 8192
TE = T // E


def _fc1_body(x_ref, w1_ref, h_ref):
    h = lax.dot_general(x_ref[...], w1_ref[0], (((1,), (1,)), ((), ())),
                        preferred_element_type=jnp.float32)
    h_ref[...] = jax.nn.gelu(h).astype(jnp.bfloat16)


def _rne_bf16_hi(b):
    return b + jnp.int32(0x7FFF) + (lax.shift_right_logical(b, 16)
                                    & jnp.int32(1))


def _fc2_body(h_ref, gate_ref, w2_ref, out_ref):
    y = lax.dot_general(h_ref[...], w2_ref[0], (((1,), (1,)), ((), ())),
                        preferred_element_type=jnp.float32)
    yg = y * gate_ref[...]
    b0 = lax.bitcast_convert_type(yg[:, :BD // 2], jnp.int32)
    b1 = lax.bitcast_convert_type(yg[:, BD // 2:], jnp.int32)
    lo = lax.shift_right_logical(_rne_bf16_hi(b0), 16)
    hi = _rne_bf16_hi(b1) & jnp.int32(-65536)
    out_ref[...] = hi | lo


BF = 512
NF1 = DFF // BF
BD = 512
ND = D // BD


def _expert_mlp(x, gate2d, W1, W2):
    h = pl.pallas_call(
        _fc1_body,
        grid=(E, NF1),
        in_specs=[
            pl.BlockSpec((TE, D), lambda e, f: (e, 0)),
            pl.BlockSpec((1, BF, D), lambda e, f: (e, f, 0)),
        ],
        out_specs=pl.BlockSpec((TE, BF), lambda e, f: (e, f)),
        out_shape=jax.ShapeDtypeStruct((T, DFF), jnp.bfloat16),
        compiler_params=pltpu.CompilerParams(
            dimension_semantics=("parallel", "arbitrary")),
    )(x, W1)
    return pl.pallas_call(
        _fc2_body,
        grid=(E, ND),
        in_specs=[
            pl.BlockSpec((TE, DFF), lambda e, d: (e, 0)),
            pl.BlockSpec((TE, 1), lambda e, d: (e, 0)),
            pl.BlockSpec((1, BD, DFF), lambda e, d: (e, d, 0)),
        ],
        out_specs=pl.BlockSpec((TE, BD // 2), lambda e, d: (e, d)),
        out_shape=jax.ShapeDtypeStruct((T, D // 2), jnp.int32),
        compiler_params=pltpu.CompilerParams(
            dimension_semantics=("parallel", "arbitrary")),
    )(h, gate2d, W2)


NW = 32
IPW = T // NW
CH = 16
NCHK = IPW // CH


def _scatter_body(res_hbm, nidx_hbm, full_hbm, nidx_v, idx_v, bufA, bufB,
                  lsA, lsB, ssA, ssB):
    w = lax.axis_index("c") * 16 + lax.axis_index("s")
    base = w * IPW
    pltpu.sync_copy(nidx_hbm.at[pl.ds(base, IPW)], nidx_v)
    for i in range(NCHK):
        v = nidx_v[pl.ds(i * CH, CH)]
        idx_v[i, ...] = ((v & 1) << 12) | lax.shift_right_logical(v, 1)
    bufs = (bufA, bufB)
    lsems = (lsA, lsB)
    ssems = (ssA, ssB)
    loads = [None] * NCHK
    scats = [None] * NCHK
    loads[0] = pltpu.async_copy(
        res_hbm.at[pl.ds(base, CH)], bufs[0], lsems[0])
    for ch in range(NCHK):
        b = ch % 2
        loads[ch].wait()
        scats[ch] = pltpu.async_copy(
            bufs[b], full_hbm.at[idx_v.at[ch]], ssems[b])
        if ch + 1 < NCHK:
            nb = (ch + 1) % 2
            if ch >= 1:
                scats[ch - 1].wait()
            loads[ch + 1] = pltpu.async_copy(
                res_hbm.at[pl.ds(base + (ch + 1) * CH, CH)], bufs[nb],
                lsems[nb])
    scats[NCHK - 2].wait()
    scats[NCHK - 1].wait()


@functools.partial(
    pl.kernel,
    out_type=jax.ShapeDtypeStruct((T, D // 2), jnp.int32),
    mesh=plsc.VectorSubcoreMesh(core_axis_name="c", subcore_axis_name="s"),
    scratch_types=[
        pltpu.VMEM((IPW,), jnp.int32),
        pltpu.VMEM((NCHK, CH), jnp.int32),
        pltpu.VMEM((CH, D // 2), jnp.int32),
        pltpu.VMEM((CH, D // 2), jnp.int32),
        pltpu.SemaphoreType.DMA,
        pltpu.SemaphoreType.DMA,
        pltpu.SemaphoreType.DMA,
        pltpu.SemaphoreType.DMA,
    ],
)
def _scatter(res_hbm, nidx_hbm, full_hbm, nidx_v, idx_v, bufA, bufB,
             lsA, lsB, ssA, ssB):
    _scatter_body(res_hbm, nidx_hbm, full_hbm, nidx_v, idx_v, bufA, bufB,
                  lsA, lsB, ssA, ssB)


BT = 256
NBT = (T // TOPK) // BT


QW = BD // 2


def _unpack_lo(a):
    return lax.bitcast_convert_type(lax.shift_left(a, 16), jnp.float32)


def _unpack_hi(a):
    return lax.bitcast_convert_type(a & jnp.int32(-65536), jnp.float32)


def _pairsum_body(a_ref, b_ref, out_ref):
    for q in range(ND):
        a = a_ref[:, pl.ds(q * QW, QW)]
        b = b_ref[:, pl.ds(q * QW, QW)]
        out_ref[:, pl.ds(q * BD, QW)] = _unpack_lo(a) + _unpack_lo(b)
        out_ref[:, pl.ds(q * BD + QW, QW)] = _unpack_hi(a) + _unpack_hi(b)


def _pairsum(full):
    return pl.pallas_call(
        _pairsum_body,
        grid=(NBT,),
        in_specs=[
            pl.BlockSpec((BT, D // 2), lambda i: (i, 0)),
            pl.BlockSpec((BT, D // 2), lambda i: (i + NBT, 0)),
        ],
        out_specs=pl.BlockSpec((BT, D), lambda i: (i, 0)),
        out_shape=jax.ShapeDtypeStruct((T // TOPK, D), jnp.float32),
        compiler_params=pltpu.CompilerParams(
            dimension_semantics=("arbitrary",)),
    )(full, full)


def kernel(inputs_shard, gate_weight, choosed_experts, new_index, W1, W2):
    gate2d = gate_weight.reshape(T, 1)
    result = _expert_mlp(inputs_shard, gate2d, W1, W2)
    full = _scatter(result, new_index)
    out2 = _pairsum(full)
    mlp_bias = jnp.zeros((D,), dtype=out2.dtype)
    return (out2, mlp_bias)


# speedup vs baseline: 1.2930x; 1.0351x over previous
import functools

import jax
import jax.numpy as jnp
from jax import lax
from jax.experimental import pallas as pl
from jax.experimental.pallas import tpu as pltpu
from jax.experimental.pallas import tpu_sc as plsc

E = 8
TOPK = 2
D = 2048
DFF = 4096
T = 8192
TE = T // E


def _fc1_body(x_ref, w1_ref, h_ref):
    h = lax.dot_general(x_ref[...], w1_ref[0], (((1,), (1,)), ((), ())),
                        preferred_element_type=jnp.float32)
    h_ref[...] = jax.nn.gelu(h).astype(jnp.bfloat16)


def _rne_bf16_hi(b):
    return b + jnp.int32(0x7FFF) + (lax.shift_right_logical(b, 16)
                                    & jnp.int32(1))


def _fc2_body(h_ref, gate_ref, w2_ref, out_ref):
    y = lax.dot_general(h_ref[...], w2_ref[0], (((1,), (1,)), ((), ())),
                        preferred_element_type=jnp.float32)
    yg = y * gate_ref[...]
    b0 = lax.bitcast_convert_type(yg[:, :BD // 2], jnp.int32)
    b1 = lax.bitcast_convert_type(yg[:, BD // 2:], jnp.int32)
    lo = lax.shift_right_logical(_rne_bf16_hi(b0), 16)
    hi = _rne_bf16_hi(b1) & jnp.int32(-65536)
    out_ref[...] = hi | lo


BF = 1024
NF1 = DFF // BF
BD = 512
ND = D // BD


def _expert_mlp(x, gate2d, W1, W2):
    h = pl.pallas_call(
        _fc1_body,
        grid=(E, NF1),
        in_specs=[
            pl.BlockSpec((TE, D), lambda e, f: (e, 0)),
            pl.BlockSpec((1, BF, D), lambda e, f: (e, f, 0)),
        ],
        out_specs=pl.BlockSpec((TE, BF), lambda e, f: (e, f)),
        out_shape=jax.ShapeDtypeStruct((T, DFF), jnp.bfloat16),
        compiler_params=pltpu.CompilerParams(
            dimension_semantics=("parallel", "arbitrary")),
    )(x, W1)
    return pl.pallas_call(
        _fc2_body,
        grid=(E, ND),
        in_specs=[
            pl.BlockSpec((TE, DFF), lambda e, d: (e, 0)),
            pl.BlockSpec((TE, 1), lambda e, d: (e, 0)),
            pl.BlockSpec((1, BD, DFF), lambda e, d: (e, d, 0)),
        ],
        out_specs=pl.BlockSpec((TE, BD // 2), lambda e, d: (e, d)),
        out_shape=jax.ShapeDtypeStruct((T, D // 2), jnp.int32),
        compiler_params=pltpu.CompilerParams(
            dimension_semantics=("parallel", "arbitrary")),
    )(h, gate2d, W2)


NW = 32
IPW = T // NW
CH = 16
NCHK = IPW // CH


def _scatter_body(res_hbm, nidx_hbm, full_hbm, nidx_v, idx_v, bufA, bufB,
                  lsA, lsB, ssA, ssB):
    w = lax.axis_index("c") * 16 + lax.axis_index("s")
    base = w * IPW
    pltpu.sync_copy(nidx_hbm.at[pl.ds(base, IPW)], nidx_v)
    for i in range(NCHK):
        v = nidx_v[pl.ds(i * CH, CH)]
        idx_v[i, ...] = ((v & 1) << 12) | lax.shift_right_logical(v, 1)
    bufs = (bufA, bufB)
    lsems = (lsA, lsB)
    ssems = (ssA, ssB)
    loads = [None] * NCHK
    scats = [None] * NCHK
    loads[0] = pltpu.async_copy(
        res_hbm.at[pl.ds(base, CH)], bufs[0], lsems[0])
    for ch in range(NCHK):
        b = ch % 2
        loads[ch].wait()
        scats[ch] = pltpu.async_copy(
            bufs[b], full_hbm.at[idx_v.at[ch]], ssems[b])
        if ch + 1 < NCHK:
            nb = (ch + 1) % 2
            if ch >= 1:
                scats[ch - 1].wait()
            loads[ch + 1] = pltpu.async_copy(
                res_hbm.at[pl.ds(base + (ch + 1) * CH, CH)], bufs[nb],
                lsems[nb])
    scats[NCHK - 2].wait()
    scats[NCHK - 1].wait()


@functools.partial(
    pl.kernel,
    out_type=jax.ShapeDtypeStruct((T, D // 2), jnp.int32),
    mesh=plsc.VectorSubcoreMesh(core_axis_name="c", subcore_axis_name="s"),
    scratch_types=[
        pltpu.VMEM((IPW,), jnp.int32),
        pltpu.VMEM((NCHK, CH), jnp.int32),
        pltpu.VMEM((CH, D // 2), jnp.int32),
        pltpu.VMEM((CH, D // 2), jnp.int32),
        pltpu.SemaphoreType.DMA,
        pltpu.SemaphoreType.DMA,
        pltpu.SemaphoreType.DMA,
        pltpu.SemaphoreType.DMA,
    ],
)
def _scatter(res_hbm, nidx_hbm, full_hbm, nidx_v, idx_v, bufA, bufB,
             lsA, lsB, ssA, ssB):
    _scatter_body(res_hbm, nidx_hbm, full_hbm, nidx_v, idx_v, bufA, bufB,
                  lsA, lsB, ssA, ssB)


BT = 256
NBT = (T // TOPK) // BT


QW = BD // 2


def _unpack_lo(a):
    return lax.bitcast_convert_type(lax.shift_left(a, 16), jnp.float32)


def _unpack_hi(a):
    return lax.bitcast_convert_type(a & jnp.int32(-65536), jnp.float32)


def _pairsum_body(a_ref, b_ref, out_ref):
    for q in range(ND):
        a = a_ref[:, pl.ds(q * QW, QW)]
        b = b_ref[:, pl.ds(q * QW, QW)]
        out_ref[:, pl.ds(q * BD, QW)] = _unpack_lo(a) + _unpack_lo(b)
        out_ref[:, pl.ds(q * BD + QW, QW)] = _unpack_hi(a) + _unpack_hi(b)


def _pairsum(full):
    return pl.pallas_call(
        _pairsum_body,
        grid=(NBT,),
        in_specs=[
            pl.BlockSpec((BT, D // 2), lambda i: (i, 0)),
            pl.BlockSpec((BT, D // 2), lambda i: (i + NBT, 0)),
        ],
        out_specs=pl.BlockSpec((BT, D), lambda i: (i, 0)),
        out_shape=jax.ShapeDtypeStruct((T // TOPK, D), jnp.float32),
        compiler_params=pltpu.CompilerParams(
            dimension_semantics=("arbitrary",)),
    )(full, full)


def kernel(inputs_shard, gate_weight, choosed_experts, new_index, W1, W2):
    gate2d = gate_weight.reshape(T, 1)
    result = _expert_mlp(inputs_shard, gate2d, W1, W2)
    full = _scatter(result, new_index)
    out2 = _pairsum(full)
    mlp_bias = jnp.zeros((D,), dtype=out2.dtype)
    return (out2, mlp_bias)


# speedup vs baseline: 1.3282x; 1.0272x over previous
import functools

import jax
import jax.numpy as jnp
from jax import lax
from jax.experimental import pallas as pl
from jax.experimental.pallas import tpu as pltpu
from jax.experimental.pallas import tpu_sc as plsc

E = 8
TOPK = 2
D = 2048
DFF = 4096
T = 8192
TE = T // E


def _fc1_body(x_ref, w1_ref, h_ref):
    h = lax.dot_general(x_ref[...], w1_ref[0], (((1,), (1,)), ((), ())),
                        preferred_element_type=jnp.float32)
    h_ref[...] = jax.nn.gelu(h).astype(jnp.bfloat16)


def _rne_bf16_hi(b):
    return b + jnp.int32(0x7FFF) + (lax.shift_right_logical(b, 16)
                                    & jnp.int32(1))


def _fc2_body(h_ref, gate_ref, w2_ref, out_ref):
    y = lax.dot_general(h_ref[...], w2_ref[0], (((1,), (1,)), ((), ())),
                        preferred_element_type=jnp.float32)
    yg = y * gate_ref[...]
    b0 = lax.bitcast_convert_type(yg[:, :BD // 2], jnp.int32)
    b1 = lax.bitcast_convert_type(yg[:, BD // 2:], jnp.int32)
    lo = lax.shift_right_logical(_rne_bf16_hi(b0), 16)
    hi = _rne_bf16_hi(b1) & jnp.int32(-65536)
    out_ref[...] = hi | lo


BF = 2048
NF1 = DFF // BF
BD = 512
ND = D // BD


def _expert_mlp(x, gate2d, W1, W2):
    h = pl.pallas_call(
        _fc1_body,
        grid=(E, NF1),
        in_specs=[
            pl.BlockSpec((TE, D), lambda e, f: (e, 0)),
            pl.BlockSpec((1, BF, D), lambda e, f: (e, f, 0)),
        ],
        out_specs=pl.BlockSpec((TE, BF), lambda e, f: (e, f)),
        out_shape=jax.ShapeDtypeStruct((T, DFF), jnp.bfloat16),
        compiler_params=pltpu.CompilerParams(
            dimension_semantics=("parallel", "arbitrary")),
    )(x, W1)
    return pl.pallas_call(
        _fc2_body,
        grid=(E, ND),
        in_specs=[
            pl.BlockSpec((TE, DFF), lambda e, d: (e, 0)),
            pl.BlockSpec((TE, 1), lambda e, d: (e, 0)),
            pl.BlockSpec((1, BD, DFF), lambda e, d: (e, d, 0)),
        ],
        out_specs=pl.BlockSpec((TE, BD // 2), lambda e, d: (e, d)),
        out_shape=jax.ShapeDtypeStruct((T, D // 2), jnp.int32),
        compiler_params=pltpu.CompilerParams(
            dimension_semantics=("parallel", "arbitrary")),
    )(h, gate2d, W2)


NW = 32
IPW = T // NW
CH = 16
NCHK = IPW // CH


NBUF = 4


def _scatter_body(res_hbm, nidx_hbm, full_hbm, nidx_v, idx_v, *bufsem):
    bufs = bufsem[:NBUF]
    lsems = bufsem[NBUF:2 * NBUF]
    ssems = bufsem[2 * NBUF:]
    w = lax.axis_index("c") * 16 + lax.axis_index("s")
    base = w * IPW
    pltpu.sync_copy(nidx_hbm.at[pl.ds(base, IPW)], nidx_v)
    for i in range(NCHK):
        v = nidx_v[pl.ds(i * CH, CH)]
        idx_v[i, ...] = ((v & 1) << 12) | lax.shift_right_logical(v, 1)
    loads = [None] * NCHK
    scats = [None] * NCHK
    for ch in range(min(NBUF, NCHK)):
        loads[ch] = pltpu.async_copy(
            res_hbm.at[pl.ds(base + ch * CH, CH)], bufs[ch], lsems[ch])
    for ch in range(NCHK):
        b = ch % NBUF
        loads[ch].wait()
        scats[ch] = pltpu.async_copy(
            bufs[b], full_hbm.at[idx_v.at[ch]], ssems[b])
        nxt = ch + 1
        if NBUF <= nxt < NCHK:
            scats[nxt - NBUF].wait()
            loads[nxt] = pltpu.async_copy(
                res_hbm.at[pl.ds(base + nxt * CH, CH)], bufs[nxt % NBUF],
                lsems[nxt % NBUF])
    for ch in range(max(0, NCHK - NBUF), NCHK):
        scats[ch].wait()


@functools.partial(
    pl.kernel,
    out_type=jax.ShapeDtypeStruct((T, D // 2), jnp.int32),
    mesh=plsc.VectorSubcoreMesh(core_axis_name="c", subcore_axis_name="s"),
    scratch_types=(
        [pltpu.VMEM((IPW,), jnp.int32),
         pltpu.VMEM((NCHK, CH), jnp.int32)]
        + [pltpu.VMEM((CH, D // 2), jnp.int32) for _ in range(NBUF)]
        + [pltpu.SemaphoreType.DMA for _ in range(2 * NBUF)]
    ),
)
def _scatter(res_hbm, nidx_hbm, full_hbm, nidx_v, idx_v, *bufsem):
    _scatter_body(res_hbm, nidx_hbm, full_hbm, nidx_v, idx_v, *bufsem)


BT = 256
NBT = (T // TOPK) // BT


QW = BD // 2


def _unpack_lo(a):
    return lax.bitcast_convert_type(lax.shift_left(a, 16), jnp.float32)


def _unpack_hi(a):
    return lax.bitcast_convert_type(a & jnp.int32(-65536), jnp.float32)


def _pairsum_body(a_ref, b_ref, out_ref):
    for q in range(ND):
        a = a_ref[:, pl.ds(q * QW, QW)]
        b = b_ref[:, pl.ds(q * QW, QW)]
        out_ref[:, pl.ds(q * BD, QW)] = _unpack_lo(a) + _unpack_lo(b)
        out_ref[:, pl.ds(q * BD + QW, QW)] = _unpack_hi(a) + _unpack_hi(b)


def _pairsum(full):
    return pl.pallas_call(
        _pairsum_body,
        grid=(NBT,),
        in_specs=[
            pl.BlockSpec((BT, D // 2), lambda i: (i, 0)),
            pl.BlockSpec((BT, D // 2), lambda i: (i + NBT, 0)),
        ],
        out_specs=pl.BlockSpec((BT, D), lambda i: (i, 0)),
        out_shape=jax.ShapeDtypeStruct((T // TOPK, D), jnp.float32),
        compiler_params=pltpu.CompilerParams(
            dimension_semantics=("arbitrary",)),
    )(full, full)


def kernel(inputs_shard, gate_weight, choosed_experts, new_index, W1, W2):
    gate2d = gate_weight.reshape(T, 1)
    result = _expert_mlp(inputs_shard, gate2d, W1, W2)
    full = _scatter(result, new_index)
    out2 = _pairsum(full)
    mlp_bias = jnp.zeros((D,), dtype=out2.dtype)
    return (out2, mlp_bias)


# speedup vs baseline: 1.3468x; 1.0140x over previous
import functools

import jax
import jax.numpy as jnp
from jax import lax
from jax.experimental import pallas as pl
from jax.experimental.pallas import tpu as pltpu
from jax.experimental.pallas import tpu_sc as plsc

E = 8
TOPK = 2
D = 2048
DFF = 4096
T = 8192
TE = T // E


def _fc1_body(x_ref, w1_ref, h_ref):
    h = lax.dot_general(x_ref[...], w1_ref[0], (((1,), (1,)), ((), ())),
                        preferred_element_type=jnp.float32)
    h_ref[...] = jax.nn.gelu(h).astype(jnp.bfloat16)


def _rne_bf16_hi(b):
    return b + jnp.int32(0x7FFF) + (lax.shift_right_logical(b, 16)
                                    & jnp.int32(1))


def _fc2_body(h_ref, gate_ref, w2_ref, out_ref):
    y = lax.dot_general(h_ref[...], w2_ref[0], (((1,), (1,)), ((), ())),
                        preferred_element_type=jnp.float32)
    yg = y * gate_ref[...]
    b0 = lax.bitcast_convert_type(yg[:, :BD // 2], jnp.int32)
    b1 = lax.bitcast_convert_type(yg[:, BD // 2:], jnp.int32)
    lo = lax.shift_right_logical(_rne_bf16_hi(b0), 16)
    hi = _rne_bf16_hi(b1) & jnp.int32(-65536)
    out_ref[...] = hi | lo


BF = 2048
NF1 = DFF // BF
BD = 512
ND = D // BD


def _expert_mlp(x, gate2d, W1, W2):
    h = pl.pallas_call(
        _fc1_body,
        grid=(E, NF1),
        in_specs=[
            pl.BlockSpec((TE, D), lambda e, f: (e, 0)),
            pl.BlockSpec((1, BF, D), lambda e, f: (e, f, 0)),
        ],
        out_specs=pl.BlockSpec((TE, BF), lambda e, f: (e, f)),
        out_shape=jax.ShapeDtypeStruct((T, DFF), jnp.bfloat16),
        compiler_params=pltpu.CompilerParams(
            dimension_semantics=("parallel", "arbitrary")),
    )(x, W1)
    return pl.pallas_call(
        _fc2_body,
        grid=(E, ND),
        in_specs=[
            pl.BlockSpec((TE, DFF), lambda e, d: (e, 0)),
            pl.BlockSpec((TE, 1), lambda e, d: (e, 0)),
            pl.BlockSpec((1, BD, DFF), lambda e, d: (e, d, 0)),
        ],
        out_specs=pl.BlockSpec((TE, BD // 2), lambda e, d: (e, d)),
        out_shape=jax.ShapeDtypeStruct((T, D // 2), jnp.int32),
        compiler_params=pltpu.CompilerParams(
            dimension_semantics=("parallel", "arbitrary")),
    )(h, gate2d, W2)


NW = 32
IPW = T // NW
CH = 16
NCHK = IPW // CH


NBUF = 6


def _scatter_body(res_hbm, nidx_hbm, full_hbm, nidx_v, idx_v, *bufsem):
    bufs = bufsem[:NBUF]
    lsems = bufsem[NBUF:2 * NBUF]
    ssems = bufsem[2 * NBUF:]
    w = lax.axis_index("c") * 16 + lax.axis_index("s")
    base = w * IPW
    pltpu.sync_copy(nidx_hbm.at[pl.ds(base, IPW)], nidx_v)
    for i in range(NCHK):
        v = nidx_v[pl.ds(i * CH, CH)]
        idx_v[i, ...] = ((v & 1) << 12) | lax.shift_right_logical(v, 1)
    loads = [None] * NCHK
    scats = [None] * NCHK
    for ch in range(min(NBUF, NCHK)):
        loads[ch] = pltpu.async_copy(
            res_hbm.at[pl.ds(base + ch * CH, CH)], bufs[ch], lsems[ch])
    for ch in range(NCHK):
        b = ch % NBUF
        loads[ch].wait()
        scats[ch] = pltpu.async_copy(
            bufs[b], full_hbm.at[idx_v.at[ch]], ssems[b])
        nxt = ch + 1
        if NBUF <= nxt < NCHK:
            scats[nxt - NBUF].wait()
            loads[nxt] = pltpu.async_copy(
                res_hbm.at[pl.ds(base + nxt * CH, CH)], bufs[nxt % NBUF],
                lsems[nxt % NBUF])
    for ch in range(max(0, NCHK - NBUF), NCHK):
        scats[ch].wait()


@functools.partial(
    pl.kernel,
    out_type=jax.ShapeDtypeStruct((T, D // 2), jnp.int32),
    mesh=plsc.VectorSubcoreMesh(core_axis_name="c", subcore_axis_name="s"),
    scratch_types=(
        [pltpu.VMEM((IPW,), jnp.int32),
         pltpu.VMEM((NCHK, CH), jnp.int32)]
        + [pltpu.VMEM((CH, D // 2), jnp.int32) for _ in range(NBUF)]
        + [pltpu.SemaphoreType.DMA for _ in range(2 * NBUF)]
    ),
)
def _scatter(res_hbm, nidx_hbm, full_hbm, nidx_v, idx_v, *bufsem):
    _scatter_body(res_hbm, nidx_hbm, full_hbm, nidx_v, idx_v, *bufsem)


BT = 512
NBT = (T // TOPK) // BT


QW = BD // 2


def _unpack_lo(a):
    return lax.bitcast_convert_type(lax.shift_left(a, 16), jnp.float32)


def _unpack_hi(a):
    return lax.bitcast_convert_type(a & jnp.int32(-65536), jnp.float32)


def _pairsum_body(a_ref, b_ref, out_ref):
    for q in range(ND):
        a = a_ref[:, pl.ds(q * QW, QW)]
        b = b_ref[:, pl.ds(q * QW, QW)]
        out_ref[:, pl.ds(q * BD, QW)] = _unpack_lo(a) + _unpack_lo(b)
        out_ref[:, pl.ds(q * BD + QW, QW)] = _unpack_hi(a) + _unpack_hi(b)


def _pairsum(full):
    return pl.pallas_call(
        _pairsum_body,
        grid=(NBT,),
        in_specs=[
            pl.BlockSpec((BT, D // 2), lambda i: (i, 0)),
            pl.BlockSpec((BT, D // 2), lambda i: (i + NBT, 0)),
        ],
        out_specs=pl.BlockSpec((BT, D), lambda i: (i, 0)),
        out_shape=jax.ShapeDtypeStruct((T // TOPK, D), jnp.float32),
        compiler_params=pltpu.CompilerParams(
            dimension_semantics=("arbitrary",)),
    )(full, full)


def kernel(inputs_shard, gate_weight, choosed_experts, new_index, W1, W2):
    gate2d = gate_weight.reshape(T, 1)
    result = _expert_mlp(inputs_shard, gate2d, W1, W2)
    full = _scatter(result, new_index)
    out2 = _pairsum(full)
    mlp_bias = jnp.zeros((D,), dtype=out2.dtype)
    return (out2, mlp_bias)
